# baseline (speedup 1.0000x reference)
import functools

import jax
import jax.numpy as jnp
from jax import lax
from jax.experimental import pallas as pl
from jax.experimental.pallas import tpu as pltpu
from jax.experimental.pallas import tpu_sc as plsc

NC = 2
NS = 16
L = 16
NW = NC * NS

BATCH = 4096
SEQ = 200
VOCAB = 1000000
FEAT = 64
N = BATCH * SEQ
R_PER_W = N // NW
U = 128
UNITS = R_PER_W // U
NPAIR = UNITS // 2
IDX_BLK = 16
PREP_BLK = 1024


def _prep_body(x_ref, o_ref):
    eye = jnp.eye(FEAT, dtype=jnp.float32)
    o_ref[:, :FEAT] = jax.lax.dot_general(
        x_ref[...], eye, (((0,), (0,)), ((), ())),
        preferred_element_type=jnp.float32)


def _prep(tabt):
    return pl.pallas_call(
        _prep_body,
        grid=(pl.cdiv(VOCAB, PREP_BLK),),
        in_specs=[pl.BlockSpec((FEAT, PREP_BLK), lambda i: (0, i))],
        out_specs=pl.BlockSpec((PREP_BLK, 128), lambda i: (i, 0)),
        out_shape=jax.ShapeDtypeStruct((VOCAB, 128), jnp.float32),
    )(tabt)


def _body(tab, toks, pos, out, idx_v, pos_v, b0buf, b1buf, gs0, gs1, ws0, ws1):
    wid = lax.axis_index("s") * NC + lax.axis_index("c")
    row0 = pl.multiple_of(wid * R_PER_W, R_PER_W)
    pltpu.sync_copy(pos, pos_v)

    def stage_idx(blk):
        off = pl.multiple_of(row0 // U + blk * IDX_BLK, 8)
        pltpu.sync_copy(toks.at[pl.ds(off, IDX_BLK)], idx_v)

    def issue_gather(u, buf, sem):
        pltpu.async_copy(tab.at[idx_v.at[lax.rem(u, IDX_BLK)]], buf, sem)

    def wait_gather(u, buf, sem):
        pltpu.make_async_copy(
            tab.at[idx_v.at[lax.rem(u, IDX_BLK)]], buf, sem).wait()

    def issue_write(u, buf, sem):
        pltpu.async_copy(buf, out.at[pl.ds(row0 + u * U, U)], sem)

    def wait_write(buf, sem):
        pltpu.make_async_copy(buf, out.at[pl.ds(row0, U)], sem).wait()

    def add_pos(u, buf):
        def row(r, p):
            for j in range(FEAT // L):
                sl = pl.ds(j * L, L)
                buf[r, sl] = buf[r, sl] + pos_v[p, sl]
            return lax.select(p + 1 == SEQ, 0, p + 1)
        lax.fori_loop(0, U, row, lax.rem(u * U, SEQ), unroll=4)

    stage_idx(0)
    issue_gather(0, b0buf, gs0)

    def step(i, _):
        a = 2 * i

        @pl.when(i > 0)
        def _w1():
            wait_write(b1buf, ws1)

        issue_gather(a + 1, b1buf, gs1)
        wait_gather(a, b0buf, gs0)
        add_pos(a, b0buf)
        issue_write(a, b0buf, ws0)

        @pl.when(lax.rem(a + 2, IDX_BLK) == 0)
        def _stage():
            stage_idx((a + 2) // IDX_BLK)

        wait_write(b0buf, ws0)
        issue_gather(a + 2, b0buf, gs0)
        wait_gather(a + 1, b1buf, gs1)
        add_pos(a + 1, b1buf)
        issue_write(a + 1, b1buf, ws1)
        return _

    lax.fori_loop(0, NPAIR - 1, step, 0, unroll=False)
    a = 2 * (NPAIR - 1)
    wait_write(b1buf, ws1)
    issue_gather(a + 1, b1buf, gs1)
    wait_gather(a, b0buf, gs0)
    add_pos(a, b0buf)
    issue_write(a, b0buf, ws0)
    wait_gather(a + 1, b1buf, gs1)
    add_pos(a + 1, b1buf)
    issue_write(a + 1, b1buf, ws1)
    wait_write(b0buf, ws0)
    wait_write(b1buf, ws1)


@jax.jit
def _encode(tab128, toks2d, pos2d):
    kern = functools.partial(
        pl.kernel,
        out_type=jax.ShapeDtypeStruct((N, 128), jnp.float32),
        mesh=plsc.VectorSubcoreMesh(core_axis_name="c", subcore_axis_name="s"),
        scratch_types=[
            pltpu.VMEM((IDX_BLK, 128), jnp.int32),
            pltpu.VMEM((SEQ, FEAT), jnp.float32),
            pltpu.VMEM((U, 128), jnp.float32),
            pltpu.VMEM((U, 128), jnp.float32),
            pltpu.SemaphoreType.DMA,
            pltpu.SemaphoreType.DMA,
            pltpu.SemaphoreType.DMA,
            pltpu.SemaphoreType.DMA,
        ],
        compiler_params=pltpu.CompilerParams(use_tc_tiling_on_sc=True),
    )(_body)
    return kern(tab128, toks2d, pos2d)


def kernel(tokens, embed_table, pos_embedding):
    tab128 = _prep(embed_table.T)
    toks2d = tokens.astype(jnp.int32).reshape(N // 128, 128)
    out128 = _encode(tab128, toks2d, pos_embedding[0, :SEQ])
    return out128.reshape(BATCH, SEQ, 128)[:, :, :FEAT]


# speedup vs baseline: 1.4970x; 1.4970x over previous
import functools

import jax
import jax.numpy as jnp
from jax import lax
from jax.experimental import pallas as pl
from jax.experimental.pallas import tpu as pltpu
from jax.experimental.pallas import tpu_sc as plsc

NC = 2
NS = 16
L = 16
NW = NC * NS

BATCH = 4096
SEQ = 200
VOCAB = 1000000
FEAT = 64
N = BATCH * SEQ
R_PER_W = N // NW
U = 256
UNITS = R_PER_W // U
NPAIR = UNITS // 2
IDX_BLK = 16
U_PER_BLK = IDX_BLK * 128 // U
PREP_BLK = 16384


def _prep_body(x_ref, o_ref):
    eye = jnp.eye(FEAT, dtype=jnp.float32)
    o_ref[:, :FEAT] = jax.lax.dot_general(
        x_ref[...], eye, (((0,), (0,)), ((), ())),
        preferred_element_type=jnp.float32)


def _prep(tabt):
    return pl.pallas_call(
        _prep_body,
        grid=(pl.cdiv(VOCAB, PREP_BLK),),
        in_specs=[pl.BlockSpec((FEAT, PREP_BLK), lambda i: (0, i))],
        out_specs=pl.BlockSpec((PREP_BLK, 128), lambda i: (i, 0)),
        out_shape=jax.ShapeDtypeStruct((VOCAB, 128), jnp.float32),
    )(tabt)


def _body(tab, toks, pos, out, idx_v, pos_v, b0buf, b1buf, gs0, gs1, ws0, ws1):
    wid = lax.axis_index("s") * NC + lax.axis_index("c")
    row0 = pl.multiple_of(wid * R_PER_W, R_PER_W)
    pltpu.sync_copy(pos, pos_v)

    def stage_idx(blk):
        off = pl.multiple_of(row0 // 128 + blk * IDX_BLK, 8)
        pltpu.sync_copy(toks.at[pl.ds(off, IDX_BLK)], idx_v)

    def issue_gather(u, buf, sem):
        r = lax.rem(u * 2, IDX_BLK)
        pltpu.async_copy(tab.at[idx_v.at[r]], buf.at[pl.ds(0, 128)], sem)
        pltpu.async_copy(tab.at[idx_v.at[r + 1]], buf.at[pl.ds(128, 128)], sem)

    def wait_gather(u, buf, sem):
        r = lax.rem(u * 2, IDX_BLK)
        pltpu.make_async_copy(
            tab.at[idx_v.at[r]], buf.at[pl.ds(0, 128)], sem).wait()
        pltpu.make_async_copy(
            tab.at[idx_v.at[r + 1]], buf.at[pl.ds(128, 128)], sem).wait()

    def issue_write(u, buf, sem):
        pltpu.async_copy(buf, out.at[pl.ds(row0 + u * U, U)], sem)

    def wait_write(buf, sem):
        pltpu.make_async_copy(buf, out.at[pl.ds(row0, U)], sem).wait()

    def add_pos(u, buf):
        def row(r, p):
            for j in range(FEAT // L):
                sl = pl.ds(j * L, L)
                buf[r, sl] = buf[r, sl] + pos_v[p, sl]
            return lax.select(p + 1 == SEQ, 0, p + 1)
        lax.fori_loop(0, U, row, lax.rem(u * U, SEQ), unroll=8)

    stage_idx(0)
    issue_gather(0, b0buf, gs0)

    def step(i, _):
        a = 2 * i

        @pl.when(i > 0)
        def _w1():
            wait_write(b1buf, ws1)

        issue_gather(a + 1, b1buf, gs1)
        wait_gather(a, b0buf, gs0)
        add_pos(a, b0buf)
        issue_write(a, b0buf, ws0)

        @pl.when(lax.rem(a + 2, U_PER_BLK) == 0)
        def _stage():
            stage_idx((a + 2) // U_PER_BLK)

        wait_write(b0buf, ws0)
        issue_gather(a + 2, b0buf, gs0)
        wait_gather(a + 1, b1buf, gs1)
        add_pos(a + 1, b1buf)
        issue_write(a + 1, b1buf, ws1)
        return _

    lax.fori_loop(0, NPAIR - 1, step, 0, unroll=False)
    a = 2 * (NPAIR - 1)
    wait_write(b1buf, ws1)
    issue_gather(a + 1, b1buf, gs1)
    wait_gather(a, b0buf, gs0)
    add_pos(a, b0buf)
    issue_write(a, b0buf, ws0)
    wait_gather(a + 1, b1buf, gs1)
    add_pos(a + 1, b1buf)
    issue_write(a + 1, b1buf, ws1)
    wait_write(b0buf, ws0)
    wait_write(b1buf, ws1)


@jax.jit
def _encode(tab128, toks2d, pos2d):
    kern = functools.partial(
        pl.kernel,
        out_type=jax.ShapeDtypeStruct((N, 128), jnp.float32),
        mesh=plsc.VectorSubcoreMesh(core_axis_name="c", subcore_axis_name="s"),
        scratch_types=[
            pltpu.VMEM((IDX_BLK, 128), jnp.int32),
            pltpu.VMEM((SEQ, FEAT), jnp.float32),
            pltpu.VMEM((U, 128), jnp.float32),
            pltpu.VMEM((U, 128), jnp.float32),
            pltpu.SemaphoreType.DMA,
            pltpu.SemaphoreType.DMA,
            pltpu.SemaphoreType.DMA,
            pltpu.SemaphoreType.DMA,
        ],
        compiler_params=pltpu.CompilerParams(use_tc_tiling_on_sc=True),
    )(_body)
    return kern(tab128, toks2d, pos2d)


def kernel(tokens, embed_table, pos_embedding):
    tab128 = _prep(embed_table.T)
    toks2d = tokens.astype(jnp.int32).reshape(N // 128, 128)
    out128 = _encode(tab128, toks2d, pos_embedding[0, :SEQ])
    return out128.reshape(BATCH, SEQ, 128)[:, :, :FEAT]


# speedup vs baseline: 1.5670x; 1.0467x over previous
import functools

import jax
import jax.numpy as jnp
from jax import lax
from jax.experimental import pallas as pl
from jax.experimental.pallas import tpu as pltpu
from jax.experimental.pallas import tpu_sc as plsc

NC = 2
NS = 16
L = 16
NW = NC * NS

BATCH = 4096
SEQ = 200
VOCAB = 1000000
FEAT = 64
N = BATCH * SEQ
R_PER_W = N // NW
U = 128
UNITS = R_PER_W // U
NPAIR = UNITS // 2
IDX_BLK = 8
U_PER_BLK = IDX_BLK * 128 // U
PREP_BLK = 16384


def _prep_body(x_ref, o_ref):
    eye = jnp.eye(FEAT, dtype=jnp.float32)
    o_ref[:, :FEAT] = jax.lax.dot_general(
        x_ref[...], eye, (((0,), (0,)), ((), ())),
        preferred_element_type=jnp.float32)


def _prep(tabt):
    return pl.pallas_call(
        _prep_body,
        grid=(pl.cdiv(VOCAB, PREP_BLK),),
        in_specs=[pl.BlockSpec((FEAT, PREP_BLK), lambda i: (0, i))],
        out_specs=pl.BlockSpec((PREP_BLK, 128), lambda i: (i, 0)),
        out_shape=jax.ShapeDtypeStruct((VOCAB, 128), jnp.float32),
    )(tabt)


def _body(tab, toks, pos, out, idx_v, pos_v, g0, g1, o0, o1,
          gs0, gs1, ws0, ws1):
    wid = lax.axis_index("s") * NC + lax.axis_index("c")
    row0 = pl.multiple_of(wid * R_PER_W, R_PER_W)
    pltpu.sync_copy(pos, pos_v)

    def stage_idx(blk):
        off = pl.multiple_of(row0 // 128 + blk * IDX_BLK, 8)
        pltpu.sync_copy(toks.at[pl.ds(off, IDX_BLK)], idx_v)

    def issue_gather(u, buf, sem):
        pltpu.async_copy(tab.at[idx_v.at[lax.rem(u, IDX_BLK)]], buf, sem)

    def wait_gather(u, buf, sem):
        pltpu.make_async_copy(
            tab.at[idx_v.at[lax.rem(u, IDX_BLK)]], buf, sem).wait()

    def issue_write(u, buf, sem):
        pltpu.async_copy(buf, out.at[pl.ds(row0 + u * U, U)], sem)

    def wait_write(buf, sem):
        pltpu.make_async_copy(buf, out.at[pl.ds(row0, U)], sem).wait()

    def add_pos(u, gbuf, obuf):
        def row(r, p):
            for j in range(FEAT // L):
                sl = pl.ds(j * L, L)
                obuf[r, sl] = gbuf[r, sl] + pos_v[p, sl]
            return lax.select(p + 1 == SEQ, 0, p + 1)
        lax.fori_loop(0, U, row, lax.rem(u * U, SEQ), unroll=8)

    stage_idx(0)
    issue_gather(0, g0, gs0)

    def step(i, _):
        a = 2 * i
        issue_gather(a + 1, g1, gs1)
        wait_gather(a, g0, gs0)

        @pl.when(i > 0)
        def _w0():
            wait_write(o0, ws0)

        add_pos(a, g0, o0)

        @pl.when(lax.rem(a + 2, U_PER_BLK) == 0)
        def _stage():
            stage_idx((a + 2) // U_PER_BLK)

        issue_gather(a + 2, g0, gs0)
        issue_write(a, o0, ws0)
        wait_gather(a + 1, g1, gs1)

        @pl.when(i > 0)
        def _w1():
            wait_write(o1, ws1)

        add_pos(a + 1, g1, o1)
        issue_write(a + 1, o1, ws1)
        return _

    lax.fori_loop(0, NPAIR - 1, step, 0, unroll=False)
    a = 2 * (NPAIR - 1)
    issue_gather(a + 1, g1, gs1)
    wait_gather(a, g0, gs0)
    wait_write(o0, ws0)
    add_pos(a, g0, o0)
    issue_write(a, o0, ws0)
    wait_gather(a + 1, g1, gs1)
    wait_write(o1, ws1)
    add_pos(a + 1, g1, o1)
    issue_write(a + 1, o1, ws1)
    wait_write(o0, ws0)
    wait_write(o1, ws1)


@jax.jit
def _encode(tab128, toks2d, pos2d):
    kern = functools.partial(
        pl.kernel,
        out_type=jax.ShapeDtypeStruct((N, 128), jnp.float32),
        mesh=plsc.VectorSubcoreMesh(core_axis_name="c", subcore_axis_name="s"),
        scratch_types=[
            pltpu.VMEM((IDX_BLK, 128), jnp.int32),
            pltpu.VMEM((SEQ, FEAT), jnp.float32),
            pltpu.VMEM((U, 128), jnp.float32),
            pltpu.VMEM((U, 128), jnp.float32),
            pltpu.VMEM((U, 128), jnp.float32),
            pltpu.VMEM((U, 128), jnp.float32),
            pltpu.SemaphoreType.DMA,
            pltpu.SemaphoreType.DMA,
            pltpu.SemaphoreType.DMA,
            pltpu.SemaphoreType.DMA,
        ],
        compiler_params=pltpu.CompilerParams(use_tc_tiling_on_sc=True),
    )(_body)
    return kern(tab128, toks2d, pos2d)


def kernel(tokens, embed_table, pos_embedding):
    tab128 = _prep(embed_table.T)
    toks2d = tokens.astype(jnp.int32).reshape(N // 128, 128)
    out128 = _encode(tab128, toks2d, pos_embedding[0, :SEQ])
    return out128.reshape(BATCH, SEQ, 128)[:, :, :FEAT]


# speedup vs baseline: 1.5687x; 1.0011x over previous
import functools

import jax
import jax.numpy as jnp
from jax import lax
from jax.experimental import pallas as pl
from jax.experimental.pallas import tpu as pltpu
from jax.experimental.pallas import tpu_sc as plsc

NC = 2
NS = 16
L = 16
NW = NC * NS

BATCH = 4096
SEQ = 200
VOCAB = 1000000
FEAT = 64
N = BATCH * SEQ
R_PER_W = N // NW
U = 128
UNITS = R_PER_W // U
NPAIR = UNITS // 2
IDX_BLK = 8
U_PER_BLK = IDX_BLK * 128 // U
PREP_BLK = 16384


def _prep_body(x_ref, o_ref):
    eye = jnp.eye(FEAT, dtype=jnp.float32)
    o_ref[:, :FEAT] = jax.lax.dot_general(
        x_ref[...], eye, (((0,), (0,)), ((), ())),
        preferred_element_type=jnp.float32)


def _prep(tabt):
    return pl.pallas_call(
        _prep_body,
        grid=(pl.cdiv(VOCAB, PREP_BLK),),
        in_specs=[pl.BlockSpec((FEAT, PREP_BLK), lambda i: (0, i))],
        out_specs=pl.BlockSpec((PREP_BLK, 128), lambda i: (i, 0)),
        out_shape=jax.ShapeDtypeStruct((VOCAB, 128), jnp.float32),
    )(tabt)


def _body(tab, toks, pos, out, idx_v, pos_v, g0, g1, o0, o1,
          gs0, gs1, ws0, ws1):
    wid = lax.axis_index("s") * NC + lax.axis_index("c")
    row0 = pl.multiple_of(wid * R_PER_W, R_PER_W)
    pltpu.sync_copy(pos, pos_v)

    def stage_idx(blk):
        off = pl.multiple_of(row0 // 128 + blk * IDX_BLK, 8)
        pltpu.sync_copy(toks.at[pl.ds(off, IDX_BLK)], idx_v.at[lax.rem(blk, 2)])

    def _idx_row(u):
        return idx_v.at[lax.rem(u // U_PER_BLK, 2), lax.rem(u, IDX_BLK)]

    def issue_gather(u, buf, sem):
        pltpu.async_copy(tab.at[_idx_row(u)], buf, sem)

    def wait_gather(u, buf, sem):
        pltpu.make_async_copy(tab.at[_idx_row(u)], buf, sem).wait()

    def issue_write(u, buf, sem):
        pltpu.async_copy(buf, out.at[pl.ds(row0 + u * U, U)], sem)

    def wait_write(buf, sem):
        pltpu.make_async_copy(buf, out.at[pl.ds(row0, U)], sem).wait()

    def add_pos(u, gbuf, obuf):
        def row(r, p):
            for j in range(FEAT // L):
                sl = pl.ds(j * L, L)
                obuf[r, sl] = gbuf[r, sl] + pos_v[p, sl]
            return lax.select(p + 1 == SEQ, 0, p + 1)
        lax.fori_loop(0, U, row, lax.rem(u * U, SEQ), unroll=8)

    stage_idx(0)
    issue_gather(0, g0, gs0)

    def step(i, _):
        a = 2 * i
        issue_gather(a + 1, g1, gs1)
        wait_gather(a, g0, gs0)

        @pl.when(i > 0)
        def _w0():
            wait_write(o0, ws0)

        add_pos(a, g0, o0)

        @pl.when(lax.rem(a + 2, U_PER_BLK) == 0)
        def _stage():
            stage_idx((a + 2) // U_PER_BLK)

        issue_gather(a + 2, g0, gs0)
        issue_write(a, o0, ws0)
        wait_gather(a + 1, g1, gs1)

        @pl.when(i > 0)
        def _w1():
            wait_write(o1, ws1)

        add_pos(a + 1, g1, o1)
        issue_write(a + 1, o1, ws1)
        return _

    lax.fori_loop(0, NPAIR - 1, step, 0, unroll=False)
    a = 2 * (NPAIR - 1)
    issue_gather(a + 1, g1, gs1)
    wait_gather(a, g0, gs0)
    wait_write(o0, ws0)
    add_pos(a, g0, o0)
    issue_write(a, o0, ws0)
    wait_gather(a + 1, g1, gs1)
    wait_write(o1, ws1)
    add_pos(a + 1, g1, o1)
    issue_write(a + 1, o1, ws1)
    wait_write(o0, ws0)
    wait_write(o1, ws1)


@jax.jit
def _encode(tab128, toks2d, pos2d):
    kern = functools.partial(
        pl.kernel,
        out_type=jax.ShapeDtypeStruct((N, 128), jnp.float32),
        mesh=plsc.VectorSubcoreMesh(core_axis_name="c", subcore_axis_name="s"),
        scratch_types=[
            pltpu.VMEM((2, IDX_BLK, 128), jnp.int32),
            pltpu.VMEM((SEQ, FEAT), jnp.float32),
            pltpu.VMEM((U, 128), jnp.float32),
            pltpu.VMEM((U, 128), jnp.float32),
            pltpu.VMEM((U, 128), jnp.float32),
            pltpu.VMEM((U, 128), jnp.float32),
            pltpu.SemaphoreType.DMA,
            pltpu.SemaphoreType.DMA,
            pltpu.SemaphoreType.DMA,
            pltpu.SemaphoreType.DMA,
        ],
        compiler_params=pltpu.CompilerParams(use_tc_tiling_on_sc=True),
    )(_body)
    return kern(tab128, toks2d, pos2d)


def kernel(tokens, embed_table, pos_embedding):
    tab128 = _prep(embed_table.T)
    toks2d = tokens.astype(jnp.int32).reshape(N // 128, 128)
    out128 = _encode(tab128, toks2d, pos_embedding[0, :SEQ])
    return out128.reshape(BATCH, SEQ, 128)[:, :, :FEAT]
